# baseline (speedup 1.0000x reference)
import functools

import jax
import jax.numpy as jnp
import numpy as np
from jax import lax
from jax.experimental import pallas as pl
from jax.experimental.pallas import tpu as pltpu
from jax.experimental.pallas import tpu_sc as plsc

N = 10000
E = 320000
D = 128
H = 8
DK = 16

NC = 2
NS = 16
CHF = 40
EPT = E // NS
NCH = EPT // CHF

_mesh = plsc.VectorSubcoreMesh(core_axis_name="c", subcore_axis_name="s")


def _proj_body(x_ref, wa_ref, ba_ref, wc_ref, bc_ref,
               h_ref, q_ref, k_ref, v_ref):
    u = (jnp.dot(x_ref[...], wa_ref[...], preferred_element_type=jnp.float32)
         + ba_ref[...])
    h = 0.5 * u * (1.0 + lax.erf(u * np.float32(1.0 / np.sqrt(2.0))))
    h_ref[...] = h
    cat = jnp.dot(h, wc_ref[...], preferred_element_type=jnp.float32) + bc_ref[...]
    q_ref[...] = cat[:, 0:D]
    k_ref[...] = cat[:, D:2 * D]
    v_ref[...] = cat[:, 2 * D:3 * D]


def _phase_a(x, W_adapt, b_adapt, Wcat, bcat):
    BN = 1000
    grid = (N // BN,)
    return pl.pallas_call(
        _proj_body,
        grid=grid,
        in_specs=[
            pl.BlockSpec((BN, D), lambda i: (i, 0)),
            pl.BlockSpec((D, D), lambda i: (0, 0)),
            pl.BlockSpec((1, D), lambda i: (0, 0)),
            pl.BlockSpec((D, 3 * D), lambda i: (0, 0)),
            pl.BlockSpec((1, 3 * D), lambda i: (0, 0)),
        ],
        out_specs=[pl.BlockSpec((BN, D), lambda i: (i, 0))] * 4,
        out_shape=[jax.ShapeDtypeStruct((N, D), jnp.float32)] * 4,
    )(x, W_adapt, b_adapt, Wcat, bcat)


@functools.partial(
    pl.kernel,
    out_type=jax.ShapeDtypeStruct((NC, N, D), jnp.float32),
    mesh=_mesh,
    scratch_types=[
        pltpu.VMEM((CHF,), jnp.int32),
        pltpu.VMEM((CHF,), jnp.int32),
        pltpu.VMEM((CHF,), jnp.int32),
        pltpu.VMEM((CHF,), jnp.int32),
        pltpu.VMEM((CHF, D), jnp.float32),
        pltpu.VMEM((CHF, D), jnp.float32),
        pltpu.VMEM((CHF, D), jnp.float32),
        pltpu.VMEM((CHF, D), jnp.float32),
        pltpu.VMEM((CHF, D), jnp.float32),
        pltpu.VMEM((CHF, D), jnp.float32),
        pltpu.VMEM_SHARED((N, D), jnp.float32),
        pltpu.SemaphoreType.DMA,
        pltpu.SemaphoreType.DMA,
    ],
)
def _fused_kernel(q_hbm, k_hbm, v_hbm, dst_hbm, src_hbm, zeros_hbm, out_hbm,
                  di0, di1, si0, si1, qb0, qb1, kb0, kb1, vb0, vb1,
                  table, gsem0, gsem1):
    cid = lax.axis_index("c")
    sid = lax.axis_index("s")
    dis = (di0, di1)
    sis = (si0, si1)
    qbs = (qb0, qb1)
    kbs = (kb0, kb1)
    vbs = (vb0, vb1)
    gsems = (gsem0, gsem1)

    base0 = sid * EPT

    @pl.when(sid == 0)
    def _init():
        pltpu.sync_copy(zeros_hbm, table)

    plsc.subcore_barrier()

    def make_fire(with_v):
        def fire(i, b):
            base = base0 + i * CHF
            pltpu.sync_copy(dst_hbm.at[pl.ds(base, CHF)], dis[b])
            pltpu.sync_copy(src_hbm.at[pl.ds(base, CHF)], sis[b])
            pltpu.async_copy(q_hbm.at[dis[b]], qbs[b], gsems[b])
            pltpu.async_copy(k_hbm.at[sis[b]], kbs[b], gsems[b])
            if with_v:
                pltpu.async_copy(v_hbm.at[sis[b]], vbs[b], gsems[b])
        return fire

    def make_drain(with_v):
        def drain(b):
            pltpu.make_async_copy(q_hbm.at[pl.ds(0, CHF)], qbs[b], gsems[b]).wait()
            pltpu.make_async_copy(q_hbm.at[pl.ds(0, CHF)], kbs[b], gsems[b]).wait()
            if with_v:
                pltpu.make_async_copy(q_hbm.at[pl.ds(0, CHF)], vbs[b], gsems[b]).wait()
        return drain

    iota16 = lax.iota(jnp.int32, 16)
    bfly = [iota16 ^ (1 << s) for s in range(4)]

    def _hsum_exp(p):
        for idx in bfly:
            p = p + p.at[idx].get(mode='promise_in_bounds')
        return jnp.exp(p)

    def compute_msg(b):
        qbv, kbv, vbv = qbs[b], kbs[b], vbs[b]

        def group(g, carry):
            for l in range(8):
                e = g * 8 + l
                for h in range(H):
                    sl = pl.ds(h * DK, DK)
                    exh = _hsum_exp(qbv[e, sl] * kbv[e, sl])
                    qbv[e, sl] = vbv[e, sl] * exh
            return carry

        lax.fori_loop(0, CHF // 8, group, 0)
        pltpu.sync_copy(qbv, table.at[dis[b]], add=True)

    def compute_den(b):
        qbv, kbv = qbs[b], kbs[b]

        def group(g, carry):
            for l in range(8):
                e = g * 8 + l
                for h in range(H):
                    sl = pl.ds(h * DK, DK)
                    exh = _hsum_exp(qbv[e, sl] * kbv[e, sl])
                    qbv[e, sl] = exh
            return carry

        lax.fori_loop(0, CHF // 8, group, 0)
        pltpu.sync_copy(qbv, table.at[dis[b]], add=True)

    def run(with_v, compute):
        fire = make_fire(with_v)
        drain = make_drain(with_v)
        fire(0, 0)

        def outer(j, carry):
            for b in range(2):
                i = j * 2 + b
                drain(b)

                @pl.when(i + 1 < NCH)
                def _():
                    fire(i + 1, 1 - b)

                compute(b)
            return carry

        lax.fori_loop(0, NCH // 2, outer, 0)

    @pl.when(cid == 0)
    def _run_num():
        run(True, compute_msg)

    @pl.when(cid == 1)
    def _run_den():
        run(False, compute_den)

    plsc.subcore_barrier()

    @pl.when(sid == 0)
    def _dump():
        pltpu.sync_copy(table, out_hbm.at[cid])


def _epi_body(slab_ref, h_ref, wa_ref, ba_ref, al_ref, g_ref,
              b_ref, wo_ref, bo_ref, o_ref):
    num = slab_ref[0]
    den = slab_ref[1]
    agg = num / (den + 1e-16)
    out = jnp.dot(agg, wa_ref[...], preferred_element_type=jnp.float32) + ba_ref[...]
    alpha = al_ref[0, 0]
    out = out * alpha + h_ref[...] * (1.0 - alpha)
    mu = jnp.mean(out, axis=1, keepdims=True)
    var = jnp.mean((out - mu) ** 2, axis=1, keepdims=True)
    out = (out - mu) / jnp.sqrt(var + 1e-5) * g_ref[...] + b_ref[...]
    o_ref[...] = jnp.dot(out, wo_ref[...], preferred_element_type=jnp.float32) + bo_ref[...]


def _phase_e(slab, h, Wa, ba, alpha, ln_g, ln_b, W_out, b_out):
    BN = 1000
    grid = (N // BN,)
    return pl.pallas_call(
        _epi_body,
        grid=grid,
        in_specs=[
            pl.BlockSpec((NC, BN, D), lambda i: (0, i, 0)),
            pl.BlockSpec((BN, D), lambda i: (i, 0)),
            pl.BlockSpec((D, D), lambda i: (0, 0)),
            pl.BlockSpec((1, D), lambda i: (0, 0)),
            pl.BlockSpec((1, 1), lambda i: (0, 0)),
            pl.BlockSpec((1, D), lambda i: (0, 0)),
            pl.BlockSpec((1, D), lambda i: (0, 0)),
            pl.BlockSpec((D, D), lambda i: (0, 0)),
            pl.BlockSpec((1, D), lambda i: (0, 0)),
        ],
        out_specs=pl.BlockSpec((BN, D), lambda i: (i, 0)),
        out_shape=jax.ShapeDtypeStruct((N, D), jnp.float32),
    )(slab, h, Wa, ba, alpha, ln_g, ln_b, W_out, b_out)


def kernel(x, edge_index, W_adapt, b_adapt, Wk, bk, Wv, bv, Wq, bq,
           Wa, ba, rel_pri, rel_att, rel_msg, skip, ln_g, ln_b,
           W_out, b_out):
    f32 = jnp.float32
    scale = jnp.repeat(rel_pri[0] / np.sqrt(DK), DK)
    Wq_eff = Wq * scale[None, :]
    bq_eff = bq * scale
    Wk_eff = jnp.einsum('dhi,hij->dhj', Wk.reshape(D, H, DK), rel_att[0]).reshape(D, D)
    bk_eff = jnp.einsum('hi,hij->hj', bk.reshape(H, DK), rel_att[0]).reshape(D)
    Wv_eff = jnp.einsum('dhi,hij->dhj', Wv.reshape(D, H, DK), rel_msg[0]).reshape(D, D)
    bv_eff = jnp.einsum('hi,hij->hj', bv.reshape(H, DK), rel_msg[0]).reshape(D)
    Wcat = jnp.concatenate([Wq_eff, Wk_eff, Wv_eff], axis=1)
    bcat = jnp.concatenate([bq_eff, bk_eff, bv_eff])[None, :]

    h, q, k, v = _phase_a(x.astype(f32), W_adapt, b_adapt[None, :], Wcat, bcat)

    src = edge_index[0].astype(jnp.int32)
    dst = edge_index[1].astype(jnp.int32)

    slab = _fused_kernel(q, k, v, dst, src, jnp.zeros((N, D), f32))

    alpha = jax.nn.sigmoid(skip[0]).reshape(1, 1)
    return _phase_e(slab, h, Wa, ba[None, :], alpha,
                    ln_g[None, :], ln_b[None, :], W_out, b_out[None, :])


# speedup vs baseline: 1.2077x; 1.2077x over previous
import functools

import jax
import jax.numpy as jnp
import numpy as np
from jax import lax
from jax.experimental import pallas as pl
from jax.experimental.pallas import tpu as pltpu
from jax.experimental.pallas import tpu_sc as plsc

N = 10000
E = 320000
EH = E // 2
D = 128
H = 8
DK = 16

NC = 2
NS = 16
NW = NC * NS
CH = 128

_mesh = plsc.VectorSubcoreMesh(core_axis_name="c", subcore_axis_name="s")


def _head_selector():
    d = lax.broadcasted_iota(jnp.int32, (D, H), 0)
    h = lax.broadcasted_iota(jnp.int32, (D, H), 1)
    return (d // DK == h).astype(jnp.float32)


def _proj_body(x_ref, wa_ref, ba_ref, wc_ref, bc_ref,
               h_ref, q_ref, k_ref, v_ref):
    u = (jnp.dot(x_ref[...], wa_ref[...], preferred_element_type=jnp.float32)
         + ba_ref[...])
    h = 0.5 * u * (1.0 + lax.erf(u * np.float32(1.0 / np.sqrt(2.0))))
    h_ref[...] = h
    cat = jnp.dot(h, wc_ref[...], preferred_element_type=jnp.float32) + bc_ref[...]
    q_ref[...] = cat[:, 0:D]
    k_ref[...] = cat[:, D:2 * D]
    v_ref[...] = cat[:, 2 * D:3 * D]


def _phase_a(x, W_adapt, b_adapt, Wcat, bcat):
    BN = 1000
    grid = (N // BN,)
    return pl.pallas_call(
        _proj_body,
        grid=grid,
        in_specs=[
            pl.BlockSpec((BN, D), lambda i: (i, 0)),
            pl.BlockSpec((D, D), lambda i: (0, 0)),
            pl.BlockSpec((1, D), lambda i: (0, 0)),
            pl.BlockSpec((D, 3 * D), lambda i: (0, 0)),
            pl.BlockSpec((1, 3 * D), lambda i: (0, 0)),
        ],
        out_specs=[pl.BlockSpec((BN, D), lambda i: (i, 0))] * 4,
        out_shape=[jax.ShapeDtypeStruct((N, D), jnp.float32)] * 4,
    )(x, W_adapt, b_adapt, Wcat, bcat)


def _make_gather(ne):
    epw = ne // NW
    nch = epw // CH
    cht = epw - nch * CH
    npair = nch // 2
    odd = nch - npair * 2

    @functools.partial(
        pl.kernel,
        out_type=[jax.ShapeDtypeStruct((ne, D), jnp.float32)] * 3,
        mesh=_mesh,
        scratch_types=[
            pltpu.VMEM((2, CH), jnp.int32),
            pltpu.VMEM((2, CH), jnp.int32),
            pltpu.VMEM((2, CH, D), jnp.float32),
            pltpu.VMEM((2, CH, D), jnp.float32),
            pltpu.VMEM((2, CH, D), jnp.float32),
            pltpu.VMEM((cht,), jnp.int32),
            pltpu.VMEM((cht,), jnp.int32),
            pltpu.VMEM((cht, D), jnp.float32),
            pltpu.VMEM((cht, D), jnp.float32),
            pltpu.VMEM((cht, D), jnp.float32),
            pltpu.SemaphoreType.DMA,
            pltpu.SemaphoreType.DMA,
            pltpu.SemaphoreType.DMA,
        ],
    )
    def gather(q_hbm, k_hbm, v_hbm, dst_hbm, src_hbm,
               qd_out, ks_out, vs_out,
               di, si, qb, kb, vb, dit, sit, qbt, kbt, vbt,
               gsem, wsem0, wsem1):
        wid = lax.axis_index("s") * NC + lax.axis_index("c")
        base0 = wid * epw
        wsems = (wsem0, wsem1)

        def do_chunk(base, n, div, siv, qbv, kbv, vbv, wsem):
            pltpu.sync_copy(dst_hbm.at[pl.ds(base, n)], div)
            pltpu.sync_copy(src_hbm.at[pl.ds(base, n)], siv)
            c1 = pltpu.async_copy(q_hbm.at[div], qbv, gsem)
            c2 = pltpu.async_copy(k_hbm.at[siv], kbv, gsem)
            c3 = pltpu.async_copy(v_hbm.at[siv], vbv, gsem)
            c1.wait()
            c2.wait()
            c3.wait()
            pltpu.async_copy(qbv, qd_out.at[pl.ds(base, n)], wsem)
            pltpu.async_copy(kbv, ks_out.at[pl.ds(base, n)], wsem)
            pltpu.async_copy(vbv, vs_out.at[pl.ds(base, n)], wsem)

        def drain_writes(b, base, n, wsem):
            pltpu.make_async_copy(qb.at[b].at[pl.ds(0, n)],
                                  qd_out.at[pl.ds(base, n)], wsem).wait()
            pltpu.make_async_copy(kb.at[b].at[pl.ds(0, n)],
                                  ks_out.at[pl.ds(base, n)], wsem).wait()
            pltpu.make_async_copy(vb.at[b].at[pl.ds(0, n)],
                                  vs_out.at[pl.ds(base, n)], wsem).wait()

        def outer(j, carry):
            for b in range(2):
                i = j * 2 + b
                base = base0 + i * CH

                @pl.when(j > 0)
                def _():
                    drain_writes(b, base, CH, wsems[b])

                do_chunk(base, CH, di.at[b], si.at[b],
                         qb.at[b], kb.at[b], vb.at[b], wsems[b])
            return carry

        lax.fori_loop(0, npair, outer, 0)
        if odd:
            drain_writes(0, base0, CH, wsems[0])
            do_chunk(base0 + (nch - 1) * CH, CH, di.at[0], si.at[0],
                     qb.at[0], kb.at[0], vb.at[0], wsems[0])
            drain_writes(1, base0, CH, wsems[1])
            drain_writes(0, base0, CH, wsems[0])
        else:
            for b in range(2):
                drain_writes(b, base0, CH, wsems[b])
        if cht:
            tbase = base0 + nch * CH
            do_chunk(tbase, cht, dit, sit, qbt, kbt, vbt, gsem)
            pltpu.make_async_copy(qbt, qd_out.at[pl.ds(tbase, cht)], gsem).wait()
            pltpu.make_async_copy(kbt, ks_out.at[pl.ds(tbase, cht)], gsem).wait()
            pltpu.make_async_copy(vbt, vs_out.at[pl.ds(tbase, cht)], gsem).wait()

    return gather


def _edge_body(qd_ref, ks_ref, vs_ref, msg_ref, exr_ref):
    S = _head_selector()
    p = qd_ref[...] * ks_ref[...]
    t = jnp.dot(p, S, preferred_element_type=jnp.float32)
    ex = jnp.exp(t)
    exr = jnp.dot(ex, S.T, preferred_element_type=jnp.float32)
    msg_ref[...] = vs_ref[...] * exr
    exr_ref[...] = exr


def _phase_c(qd, ks, vs):
    BE = 2000
    ne = qd.shape[0]
    grid = (ne // BE,)
    return pl.pallas_call(
        _edge_body,
        grid=grid,
        in_specs=[pl.BlockSpec((BE, D), lambda i: (i, 0))] * 3,
        out_specs=[pl.BlockSpec((BE, D), lambda i: (i, 0))] * 2,
        out_shape=[jax.ShapeDtypeStruct((ne, D), jnp.float32)] * 2,
    )(qd, ks, vs)


def _make_scatter(ne):
    ept = ne // NS
    nch = ept // CH
    cht = ept - nch * CH
    npair = nch // 2
    assert nch == npair * 2

    @functools.partial(
        pl.kernel,
        out_type=jax.ShapeDtypeStruct((NC, N, D), jnp.float32),
        mesh=_mesh,
        scratch_types=[
            pltpu.VMEM((2, CH), jnp.int32),
            pltpu.VMEM((2, CH, D), jnp.float32),
            pltpu.VMEM((cht,), jnp.int32),
            pltpu.VMEM((cht, D), jnp.float32),
            pltpu.VMEM_SHARED((N, D), jnp.float32),
            pltpu.SemaphoreType.DMA,
        ],
    )
    def scatter(msg_hbm, exr_hbm, dst_hbm, zeros_hbm, out_hbm,
                di, mb, dit, mbt, table, lsem):
        cid = lax.axis_index("c")
        sid = lax.axis_index("s")
        base0 = sid * ept

        @pl.when(sid == 0)
        def _init():
            pltpu.sync_copy(zeros_hbm, table)

        plsc.subcore_barrier()

        def run(rows_hbm):
            pltpu.sync_copy(dst_hbm.at[pl.ds(base0, CH)], di.at[0])
            pltpu.async_copy(rows_hbm.at[pl.ds(base0, CH)], mb.at[0], lsem)

            def outer(j, carry):
                for b in range(2):
                    i = j * 2 + b
                    base = base0 + i * CH
                    pltpu.make_async_copy(rows_hbm.at[pl.ds(base, CH)],
                                          mb.at[b], lsem).wait()
                    nb = 1 - b
                    nxt = base + CH

                    @pl.when(i + 1 < nch)
                    def _():
                        pltpu.sync_copy(dst_hbm.at[pl.ds(nxt, CH)], di.at[nb])
                        pltpu.async_copy(rows_hbm.at[pl.ds(nxt, CH)], mb.at[nb], lsem)

                    pltpu.sync_copy(mb.at[b], table.at[di.at[b]], add=True)
                return carry

            lax.fori_loop(0, npair, outer, 0)
            if cht:
                tbase = base0 + nch * CH
                pltpu.sync_copy(dst_hbm.at[pl.ds(tbase, cht)], dit)
                pltpu.sync_copy(rows_hbm.at[pl.ds(tbase, cht)], mbt)
                pltpu.sync_copy(mbt, table.at[dit], add=True)

        @pl.when(cid == 0)
        def _run_msg():
            run(msg_hbm)

        @pl.when(cid == 1)
        def _run_den():
            run(exr_hbm)

        plsc.subcore_barrier()

        @pl.when(sid == 0)
        def _dump():
            pltpu.sync_copy(table, out_hbm.at[cid])

    return scatter


_gather_half = _make_gather(EH)
_scatter_half = _make_scatter(EH)


def _epi_body(s1_ref, s2_ref, h_ref, wa_ref, ba_ref, al_ref, g_ref, b_ref,
              wo_ref, bo_ref, o_ref):
    num = s1_ref[0] + s2_ref[0]
    den = s1_ref[1] + s2_ref[1]
    agg = num / (den + 1e-16)
    out = jnp.dot(agg, wa_ref[...], preferred_element_type=jnp.float32) + ba_ref[...]
    alpha = al_ref[0, 0]
    out = out * alpha + h_ref[...] * (1.0 - alpha)
    mu = jnp.mean(out, axis=1, keepdims=True)
    var = jnp.mean((out - mu) ** 2, axis=1, keepdims=True)
    out = (out - mu) / jnp.sqrt(var + 1e-5) * g_ref[...] + b_ref[...]
    o_ref[...] = jnp.dot(out, wo_ref[...], preferred_element_type=jnp.float32) + bo_ref[...]


def _phase_e(s1, s2, h, Wa, ba, alpha, ln_g, ln_b, W_out, b_out):
    BN = 1000
    grid = (N // BN,)
    return pl.pallas_call(
        _epi_body,
        grid=grid,
        in_specs=[
            pl.BlockSpec((NC, BN, D), lambda i: (0, i, 0)),
            pl.BlockSpec((NC, BN, D), lambda i: (0, i, 0)),
            pl.BlockSpec((BN, D), lambda i: (i, 0)),
            pl.BlockSpec((D, D), lambda i: (0, 0)),
            pl.BlockSpec((1, D), lambda i: (0, 0)),
            pl.BlockSpec((1, 1), lambda i: (0, 0)),
            pl.BlockSpec((1, D), lambda i: (0, 0)),
            pl.BlockSpec((1, D), lambda i: (0, 0)),
            pl.BlockSpec((D, D), lambda i: (0, 0)),
            pl.BlockSpec((1, D), lambda i: (0, 0)),
        ],
        out_specs=pl.BlockSpec((BN, D), lambda i: (i, 0)),
        out_shape=jax.ShapeDtypeStruct((N, D), jnp.float32),
    )(s1, s2, h, Wa, ba, alpha, ln_g, ln_b, W_out, b_out)


def kernel(x, edge_index, W_adapt, b_adapt, Wk, bk, Wv, bv, Wq, bq,
           Wa, ba, rel_pri, rel_att, rel_msg, skip, ln_g, ln_b,
           W_out, b_out):
    f32 = jnp.float32
    scale = jnp.repeat(rel_pri[0] / np.sqrt(DK), DK)
    Wq_eff = Wq * scale[None, :]
    bq_eff = bq * scale
    Wk_eff = jnp.einsum('dhi,hij->dhj', Wk.reshape(D, H, DK), rel_att[0]).reshape(D, D)
    bk_eff = jnp.einsum('hi,hij->hj', bk.reshape(H, DK), rel_att[0]).reshape(D)
    Wv_eff = jnp.einsum('dhi,hij->dhj', Wv.reshape(D, H, DK), rel_msg[0]).reshape(D, D)
    bv_eff = jnp.einsum('hi,hij->hj', bv.reshape(H, DK), rel_msg[0]).reshape(D)
    Wcat = jnp.concatenate([Wq_eff, Wk_eff, Wv_eff], axis=1)
    bcat = jnp.concatenate([bq_eff, bk_eff, bv_eff])[None, :]

    h, q, k, v = _phase_a(x.astype(f32), W_adapt, b_adapt[None, :], Wcat, bcat)

    src = edge_index[0].astype(jnp.int32)
    dst = edge_index[1].astype(jnp.int32)
    zeros = jnp.zeros((N, D), f32)

    d1, s1i = dst[0:EH], src[0:EH]
    d2, s2i = dst[EH:E], src[EH:E]
    qd1, ks1, vs1 = _gather_half(q, k, v, d1, s1i)
    qd2, ks2, vs2 = _gather_half(q, k, v, d2, s2i)
    msg1, exr1 = _phase_c(qd1, ks1, vs1)
    slab1 = _scatter_half(msg1, exr1, d1, zeros)
    msg2, exr2 = _phase_c(qd2, ks2, vs2)
    slab2 = _scatter_half(msg2, exr2, d2, zeros)

    alpha = jax.nn.sigmoid(skip[0]).reshape(1, 1)
    return _phase_e(slab1, slab2, h, Wa, ba[None, :], alpha,
                    ln_g[None, :], ln_b[None, :], W_out, b_out[None, :])
